# baseline (speedup 1.0000x reference)
import functools

import jax
import jax.numpy as jnp
from jax import lax
from jax.experimental import pallas as pl
from jax.experimental.pallas import tpu as pltpu
from jax.experimental.pallas import tpu_sc as plsc

_N_RAYS = 65536
_RES = 128
_NSAMP = 128
_NEAR = 0.1
_FAR = 3.0
_OCC_THRES = 0.01
_TAB = _RES * _RES * _RES

_SBLK = 8
_RV = _N_RAYS // 1024


def _idx_body(t_ref, ox_ref, oy_ref, oz_ref, dx_ref, dy_ref, dz_ref, idx_ref):
    j = pl.program_id(0)
    dxv, dyv, dzv = dx_ref[...], dy_ref[...], dz_ref[...]
    norm = jnp.sqrt(dxv * dxv + dyv * dyv + dzv * dzv) + 1e-8
    inv = 64.0 / norm
    ax = ox_ref[...] * 64.0 + 64.0
    ay = oy_ref[...] * 64.0 + 64.0
    az = oz_ref[...] * 64.0 + 64.0
    bx, by, bz = dxv * inv, dyv * inv, dzv * inv

    for sl in range(_SBLK):
        ts = t_ref[j * _SBLK + sl]
        ix = jnp.floor(ax + bx * ts).astype(jnp.int32)
        iy = jnp.floor(ay + by * ts).astype(jnp.int32)
        iz = jnp.floor(az + bz * ts).astype(jnp.int32)
        flat = ((ix << 14) + (iy << 7) + iz) & (_TAB - 1)
        idx_ref[pl.ds(sl * _RV, _RV)] = flat


def _compute_idx(t_mid, ox, oy, oz, dx, dy, dz):
    nblk = _NSAMP // _SBLK
    ray_spec = pl.BlockSpec((_RV, 8, 128), lambda b: (0, 0, 0))
    return pl.pallas_call(
        _idx_body,
        grid=(nblk,),
        in_specs=[pl.BlockSpec(memory_space=pltpu.SMEM)] + [ray_spec] * 6,
        out_specs=pl.BlockSpec((_SBLK * _RV, 8, 128), lambda b: (b, 0, 0)),
        out_shape=jax.ShapeDtypeStruct((_NSAMP * _RV, 8, 128), jnp.int32),
    )(t_mid, ox, oy, oz, dx, dy, dz)


def _sc_gather(table, idx_flat):
    nw = 32
    b_total = idx_flat.shape[0]
    b_per_w = b_total // nw
    ch = 16384
    nch = b_per_w // ch
    nbuf = 2
    mesh = plsc.VectorSubcoreMesh(core_axis_name="c", subcore_axis_name="s")

    @functools.partial(
        pl.kernel,
        out_type=jax.ShapeDtypeStruct((b_total,), jnp.float32),
        mesh=mesh,
        scratch_types=[
            pltpu.VMEM((ch,), jnp.int32),
            pltpu.VMEM((ch,), jnp.int32),
            pltpu.VMEM((ch,), jnp.float32),
            pltpu.VMEM((ch,), jnp.float32),
            pltpu.SemaphoreType.DMA,
            pltpu.SemaphoreType.DMA,
            pltpu.SemaphoreType.DMA,
            pltpu.SemaphoreType.DMA,
        ],
    )
    def gather_k(tab_hbm, idx_hbm, out_hbm, idx_v0, idx_v1, occ_v0, occ_v1,
                 sem_in, sem_g0, sem_g1, sem_out):
        idx_v = [idx_v0, idx_v1]
        occ_v = [occ_v0, occ_v1]
        sem_g = [sem_g0, sem_g1]
        wid = lax.axis_index("s") * 2 + lax.axis_index("c")
        base = wid * b_per_w

        def stage_in(c):
            return pltpu.async_copy(
                idx_hbm.at[pl.ds(base + c * ch, ch)], idx_v[c % nbuf],
                sem_in)

        def stage_out(c):
            return pltpu.async_copy(
                occ_v[c % nbuf], out_hbm.at[pl.ds(base + c * ch, ch)],
                sem_out)

        def gstart(c):
            b = c % nbuf
            return pltpu.async_copy(tab_hbm.at[idx_v[b]], occ_v[b],
                                    sem_g[c % 2])

        in_descs = [None] * nch
        out_descs = [None] * nch
        g_descs = [None] * nch
        in_descs[0] = stage_in(0)
        if nch > 1:
            in_descs[1] = stage_in(1)
        in_descs[0].wait()
        g_descs[0] = gstart(0)
        for c in range(1, nch):
            in_descs[c].wait()
            if c >= nbuf:
                out_descs[c - nbuf].wait()
            g_descs[c] = gstart(c)
            g_descs[c - 1].wait()
            out_descs[c - 1] = stage_out(c - 1)
            if c + 1 < nch:
                in_descs[c + 1] = stage_in(c + 1)
        g_descs[nch - 1].wait()
        out_descs[nch - 1] = stage_out(nch - 1)
        out_descs[nch - 2].wait()
        out_descs[nch - 1].wait()

    return gather_k(table, idx_flat)


def _comp_body(t_ref, dt_ref, occ_ref, ox_ref, oy_ref, oz_ref,
               dx_ref, dy_ref, dz_ref, cx_ref, cy_ref, cz_ref):
    oxv = ox_ref[...][0]
    oyv = oy_ref[...][0]
    ozv = oz_ref[...][0]
    dxv = dx_ref[...][0]
    dyv = dy_ref[...][0]
    dzv = dz_ref[...][0]
    norm = jnp.sqrt(dxv * dxv + dyv * dyv + dzv * dzv) + 1e-8
    inv = 1.0 / norm
    ndx, ndy, ndz = dxv * inv, dyv * inv, dzv * inv

    def slab(a, b):
        r = 1.0 / (b * 64.0)
        la = (0.0 - a) * r
        lb = (128.0 - a) * r
        return jnp.minimum(la, lb), jnp.maximum(la, lb)

    lox, hix = slab(oxv * 64.0 + 64.0, ndx)
    loy, hiy = slab(oyv * 64.0 + 64.0, ndy)
    loz, hiz = slab(ozv * 64.0 + 64.0, ndz)
    t_lo = jnp.maximum(jnp.maximum(lox, loy), loz)
    t_hi = jnp.minimum(jnp.minimum(hix, hiy), hiz)

    occ = occ_ref[...][:, 0]
    t3 = t_ref[...].reshape(_NSAMP, 1, 1)
    inside = (t3 >= t_lo) & (t3 <= t_hi)
    sp = jnp.log1p(jnp.exp(occ))
    sigma = jnp.where((occ > _OCC_THRES) & inside, sp, 0.0)
    cum = sigma * dt_ref[...].reshape(_NSAMP, 1, 1)
    k = 1
    while k < _NSAMP:
        z = jnp.zeros((k, 8, 128), jnp.float32)
        cum = cum + jnp.concatenate([z, cum[:-k]], axis=0)
        k *= 2
    e = jnp.exp(-cum)
    e_prev = jnp.concatenate(
        [jnp.ones((1, 8, 128), jnp.float32), e[:-1]], axis=0)
    w = e_prev - e
    wsum = 1.0 - e[_NSAMP - 1]
    tsum = jnp.sum(w * t3, axis=0)

    cx_ref[0] = oxv * wsum + ndx * tsum
    cy_ref[0] = oyv * wsum + ndy * tsum
    cz_ref[0] = ozv * wsum + ndz * tsum


def _composite(t_mid, dt, occ4, ox, oy, oz, dx, dy, dz):
    tcol_spec = pl.BlockSpec((_NSAMP, 1), lambda b: (0, 0))
    ray_spec = pl.BlockSpec((1, 8, 128), lambda b: (b, 0, 0))
    out_sds = jax.ShapeDtypeStruct((_RV, 8, 128), jnp.float32)
    return pl.pallas_call(
        _comp_body,
        grid=(_RV,),
        in_specs=[tcol_spec, tcol_spec,
                  pl.BlockSpec((_NSAMP, 1, 8, 128), lambda b: (0, b, 0, 0))]
                 + [ray_spec] * 6,
        out_specs=[ray_spec] * 3,
        out_shape=[out_sds, out_sds, out_sds],
    )(t_mid.reshape(_NSAMP, 1), dt.reshape(_NSAMP, 1),
      occ4, ox, oy, oz, dx, dy, dz)


def kernel(rays_o, rays_d, occ_grid):
    f32 = jnp.float32
    t_edges = jnp.linspace(_NEAR, _FAR, _NSAMP + 1, dtype=f32)
    t_mid = 0.5 * (t_edges[:-1] + t_edges[1:])
    dt = t_edges[1:] - t_edges[:-1]

    ox = rays_o[:, 0].reshape(_RV, 8, 128)
    oy = rays_o[:, 1].reshape(_RV, 8, 128)
    oz = rays_o[:, 2].reshape(_RV, 8, 128)
    dx = rays_d[:, 0].reshape(_RV, 8, 128)
    dy = rays_d[:, 1].reshape(_RV, 8, 128)
    dz = rays_d[:, 2].reshape(_RV, 8, 128)

    idx3 = _compute_idx(t_mid, ox, oy, oz, dx, dy, dz)
    occ = _sc_gather(occ_grid, idx3.reshape(-1))
    occ4 = occ.reshape(_NSAMP, _RV, 8, 128)
    cx, cy, cz = _composite(t_mid, dt, occ4, ox, oy, oz, dx, dy, dz)
    return jnp.stack(
        [cx.reshape(-1), cy.reshape(-1), cz.reshape(-1)], axis=-1)


# speedup vs baseline: 1.0989x; 1.0989x over previous
import functools

import jax
import jax.numpy as jnp
from jax import lax
from jax.experimental import pallas as pl
from jax.experimental.pallas import tpu as pltpu
from jax.experimental.pallas import tpu_sc as plsc

_N_RAYS = 65536
_RES = 128
_NSAMP = 128
_NEAR = 0.1
_FAR = 3.0
_OCC_THRES = 0.01
_TAB = _RES * _RES * _RES

_SBLK = 8
_RV = _N_RAYS // 1024


def _idx_body(t_ref, ox_ref, oy_ref, oz_ref, dx_ref, dy_ref, dz_ref, idx_ref):
    j = pl.program_id(0)
    dxv, dyv, dzv = dx_ref[...], dy_ref[...], dz_ref[...]
    norm = jnp.sqrt(dxv * dxv + dyv * dyv + dzv * dzv) + 1e-8
    inv = 64.0 / norm
    ax = ox_ref[...] * 64.0 + 64.0
    ay = oy_ref[...] * 64.0 + 64.0
    az = oz_ref[...] * 64.0 + 64.0
    bx, by, bz = dxv * inv, dyv * inv, dzv * inv

    rv = dx_ref.shape[0]
    for sl in range(_SBLK):
        ts = t_ref[j * _SBLK + sl]
        ix = jnp.floor(ax + bx * ts).astype(jnp.int32)
        iy = jnp.floor(ay + by * ts).astype(jnp.int32)
        iz = jnp.floor(az + bz * ts).astype(jnp.int32)
        flat = ((ix << 14) + (iy << 7) + iz) & (_TAB - 1)
        idx_ref[pl.ds(sl * rv, rv)] = flat


def _compute_idx(t_mid, ox, oy, oz, dx, dy, dz):
    rv = ox.shape[0]
    nblk = _NSAMP // _SBLK
    ray_spec = pl.BlockSpec((rv, 8, 128), lambda b: (0, 0, 0))
    return pl.pallas_call(
        _idx_body,
        grid=(nblk,),
        in_specs=[pl.BlockSpec(memory_space=pltpu.SMEM)] + [ray_spec] * 6,
        out_specs=pl.BlockSpec((_SBLK * rv, 8, 128), lambda b: (b, 0, 0)),
        out_shape=jax.ShapeDtypeStruct((_NSAMP * rv, 8, 128), jnp.int32),
    )(t_mid, ox, oy, oz, dx, dy, dz)


def _sc_gather(table, idx_flat):
    nw = 32
    b_total = idx_flat.shape[0]
    b_per_w = b_total // nw
    ch = 16384
    nch = b_per_w // ch
    nbuf = 2
    mesh = plsc.VectorSubcoreMesh(core_axis_name="c", subcore_axis_name="s")

    @functools.partial(
        pl.kernel,
        out_type=jax.ShapeDtypeStruct((b_total,), jnp.float32),
        mesh=mesh,
        scratch_types=[
            pltpu.VMEM((ch,), jnp.int32),
            pltpu.VMEM((ch,), jnp.int32),
            pltpu.VMEM((ch,), jnp.float32),
            pltpu.VMEM((ch,), jnp.float32),
            pltpu.SemaphoreType.DMA,
            pltpu.SemaphoreType.DMA,
            pltpu.SemaphoreType.DMA,
            pltpu.SemaphoreType.DMA,
        ],
    )
    def gather_k(tab_hbm, idx_hbm, out_hbm, idx_v0, idx_v1, occ_v0, occ_v1,
                 sem_in, sem_g0, sem_g1, sem_out):
        idx_v = [idx_v0, idx_v1]
        occ_v = [occ_v0, occ_v1]
        sem_g = [sem_g0, sem_g1]
        wid = lax.axis_index("s") * 2 + lax.axis_index("c")
        base = wid * b_per_w

        def stage_in(c):
            return pltpu.async_copy(
                idx_hbm.at[pl.ds(base + c * ch, ch)], idx_v[c % nbuf],
                sem_in)

        def stage_out(c):
            return pltpu.async_copy(
                occ_v[c % nbuf], out_hbm.at[pl.ds(base + c * ch, ch)],
                sem_out)

        def gstart(c):
            b = c % nbuf
            return pltpu.async_copy(tab_hbm.at[idx_v[b]], occ_v[b],
                                    sem_g[c % 2])

        in_descs = [None] * nch
        out_descs = [None] * nch
        in_descs[0] = stage_in(0)
        for c in range(nch):
            b = c % nbuf
            in_descs[c].wait()
            if c + 1 < nch:
                in_descs[c + 1] = stage_in(c + 1)
            if c >= nbuf:
                out_descs[c - nbuf].wait()
            gstart(c).wait()
            out_descs[c] = stage_out(c)
        for c in range(nch - nbuf, nch):
            out_descs[c].wait()

    return gather_k(table, idx_flat)


def _comp_body(t_ref, dt_ref, occ_ref, ox_ref, oy_ref, oz_ref,
               dx_ref, dy_ref, dz_ref, cx_ref, cy_ref, cz_ref):
    oxv = ox_ref[...][0]
    oyv = oy_ref[...][0]
    ozv = oz_ref[...][0]
    dxv = dx_ref[...][0]
    dyv = dy_ref[...][0]
    dzv = dz_ref[...][0]
    norm = jnp.sqrt(dxv * dxv + dyv * dyv + dzv * dzv) + 1e-8
    inv = 1.0 / norm
    ndx, ndy, ndz = dxv * inv, dyv * inv, dzv * inv

    def slab(a, b):
        r = 1.0 / (b * 64.0)
        la = (0.0 - a) * r
        lb = (128.0 - a) * r
        return jnp.minimum(la, lb), jnp.maximum(la, lb)

    lox, hix = slab(oxv * 64.0 + 64.0, ndx)
    loy, hiy = slab(oyv * 64.0 + 64.0, ndy)
    loz, hiz = slab(ozv * 64.0 + 64.0, ndz)
    t_lo = jnp.maximum(jnp.maximum(lox, loy), loz)
    t_hi = jnp.minimum(jnp.minimum(hix, hiy), hiz)

    occ = occ_ref[...][:, 0]
    t3 = t_ref[...].reshape(_NSAMP, 1, 1)
    inside = (t3 >= t_lo) & (t3 <= t_hi)
    sp = jnp.log1p(jnp.exp(occ))
    sigma = jnp.where((occ > _OCC_THRES) & inside, sp, 0.0)
    cum = sigma * dt_ref[...].reshape(_NSAMP, 1, 1)
    k = 1
    while k < _NSAMP:
        z = jnp.zeros((k, 8, 128), jnp.float32)
        cum = cum + jnp.concatenate([z, cum[:-k]], axis=0)
        k *= 2
    e = jnp.exp(-cum)
    e_prev = jnp.concatenate(
        [jnp.ones((1, 8, 128), jnp.float32), e[:-1]], axis=0)
    w = e_prev - e
    wsum = 1.0 - e[_NSAMP - 1]
    tsum = jnp.sum(w * t3, axis=0)

    cx_ref[0] = oxv * wsum + ndx * tsum
    cy_ref[0] = oyv * wsum + ndy * tsum
    cz_ref[0] = ozv * wsum + ndz * tsum


def _composite(t_mid, dt, occ4, ox, oy, oz, dx, dy, dz):
    rv = ox.shape[0]
    tcol_spec = pl.BlockSpec((_NSAMP, 1), lambda b: (0, 0))
    ray_spec = pl.BlockSpec((1, 8, 128), lambda b: (b, 0, 0))
    out_sds = jax.ShapeDtypeStruct((rv, 8, 128), jnp.float32)
    return pl.pallas_call(
        _comp_body,
        grid=(rv,),
        in_specs=[tcol_spec, tcol_spec,
                  pl.BlockSpec((_NSAMP, 1, 8, 128), lambda b: (0, b, 0, 0))]
                 + [ray_spec] * 6,
        out_specs=[ray_spec] * 3,
        out_shape=[out_sds, out_sds, out_sds],
    )(t_mid.reshape(_NSAMP, 1), dt.reshape(_NSAMP, 1),
      occ4, ox, oy, oz, dx, dy, dz)


def kernel(rays_o, rays_d, occ_grid):
    f32 = jnp.float32
    t_edges = jnp.linspace(_NEAR, _FAR, _NSAMP + 1, dtype=f32)
    t_mid = 0.5 * (t_edges[:-1] + t_edges[1:])
    dt = t_edges[1:] - t_edges[:-1]

    ox = rays_o[:, 0].reshape(_RV, 8, 128)
    oy = rays_o[:, 1].reshape(_RV, 8, 128)
    oz = rays_o[:, 2].reshape(_RV, 8, 128)
    dx = rays_d[:, 0].reshape(_RV, 8, 128)
    dy = rays_d[:, 1].reshape(_RV, 8, 128)
    dz = rays_d[:, 2].reshape(_RV, 8, 128)

    h = _RV // 2
    outs = []
    halves = []
    for p in range(2):
        sl = slice(p * h, (p + 1) * h)
        args = (ox[sl], oy[sl], oz[sl], dx[sl], dy[sl], dz[sl])
        idx3 = _compute_idx(t_mid, *args)
        occ = _sc_gather(occ_grid, idx3.reshape(-1))
        halves.append((occ.reshape(_NSAMP, h, 8, 128), args))
    for occ4, args in halves:
        outs.append(_composite(t_mid, dt, occ4, *args))
    cx = jnp.concatenate([outs[0][0], outs[1][0]]).reshape(-1)
    cy = jnp.concatenate([outs[0][1], outs[1][1]]).reshape(-1)
    cz = jnp.concatenate([outs[0][2], outs[1][2]]).reshape(-1)
    return jnp.stack([cx, cy, cz], axis=-1)


# speedup vs baseline: 1.1070x; 1.0073x over previous
import functools

import jax
import jax.numpy as jnp
from jax import lax
from jax.experimental import pallas as pl
from jax.experimental.pallas import tpu as pltpu
from jax.experimental.pallas import tpu_sc as plsc

_N_RAYS = 65536
_RES = 128
_NSAMP = 128
_NEAR = 0.1
_FAR = 3.0
_OCC_THRES = 0.01
_TAB = _RES * _RES * _RES

_SBLK = 8
_RV = _N_RAYS // 1024


def _idx_body(t_ref, ox_ref, oy_ref, oz_ref, dx_ref, dy_ref, dz_ref, idx_ref):
    j = pl.program_id(0)
    dxv, dyv, dzv = dx_ref[...], dy_ref[...], dz_ref[...]
    norm = jnp.sqrt(dxv * dxv + dyv * dyv + dzv * dzv) + 1e-8
    inv = 64.0 / norm
    ax = ox_ref[...] * 64.0 + 64.0
    ay = oy_ref[...] * 64.0 + 64.0
    az = oz_ref[...] * 64.0 + 64.0
    bx, by, bz = dxv * inv, dyv * inv, dzv * inv

    rv = dx_ref.shape[0]
    for sl in range(_SBLK):
        ts = t_ref[j * _SBLK + sl]
        ix = jnp.floor(ax + bx * ts).astype(jnp.int32)
        iy = jnp.floor(ay + by * ts).astype(jnp.int32)
        iz = jnp.floor(az + bz * ts).astype(jnp.int32)
        flat = ((ix << 14) + (iy << 7) + iz) & (_TAB - 1)
        idx_ref[pl.ds(sl * rv, rv)] = flat


def _compute_idx(t_mid, ox, oy, oz, dx, dy, dz):
    rv = ox.shape[0]
    nblk = _NSAMP // _SBLK
    ray_spec = pl.BlockSpec((rv, 8, 128), lambda b: (0, 0, 0))
    return pl.pallas_call(
        _idx_body,
        grid=(nblk,),
        in_specs=[pl.BlockSpec(memory_space=pltpu.SMEM)] + [ray_spec] * 6,
        out_specs=pl.BlockSpec((_SBLK * rv, 8, 128), lambda b: (b, 0, 0)),
        out_shape=jax.ShapeDtypeStruct((_NSAMP * rv, 8, 128), jnp.int32),
    )(t_mid, ox, oy, oz, dx, dy, dz)


def _sc_gather(table, idx_flat):
    nw = 32
    b_total = idx_flat.shape[0]
    b_per_w = b_total // nw
    ch = 16384
    nch = b_per_w // ch
    nbuf = 2
    mesh = plsc.VectorSubcoreMesh(core_axis_name="c", subcore_axis_name="s")

    @functools.partial(
        pl.kernel,
        out_type=jax.ShapeDtypeStruct((b_total,), jnp.float32),
        mesh=mesh,
        scratch_types=[
            pltpu.VMEM((ch,), jnp.int32),
            pltpu.VMEM((ch,), jnp.int32),
            pltpu.VMEM((ch,), jnp.float32),
            pltpu.VMEM((ch,), jnp.float32),
            pltpu.SemaphoreType.DMA,
            pltpu.SemaphoreType.DMA,
            pltpu.SemaphoreType.DMA,
            pltpu.SemaphoreType.DMA,
        ],
    )
    def gather_k(tab_hbm, idx_hbm, out_hbm, idx_v0, idx_v1, occ_v0, occ_v1,
                 sem_in, sem_g0, sem_g1, sem_out):
        idx_v = [idx_v0, idx_v1]
        occ_v = [occ_v0, occ_v1]
        sem_g = [sem_g0, sem_g1]
        wid = lax.axis_index("s") * 2 + lax.axis_index("c")
        base = wid * b_per_w

        def stage_in(c):
            return pltpu.async_copy(
                idx_hbm.at[pl.ds(base + c * ch, ch)], idx_v[c % nbuf],
                sem_in)

        def stage_out(c):
            return pltpu.async_copy(
                occ_v[c % nbuf], out_hbm.at[pl.ds(base + c * ch, ch)],
                sem_out)

        def gstart(c):
            b = c % nbuf
            return pltpu.async_copy(tab_hbm.at[idx_v[b]], occ_v[b],
                                    sem_g[c % 2])

        in_descs = [None] * nch
        out_descs = [None] * nch
        in_descs[0] = stage_in(0)
        for c in range(nch):
            b = c % nbuf
            in_descs[c].wait()
            if c + 1 < nch:
                in_descs[c + 1] = stage_in(c + 1)
            if c >= nbuf:
                out_descs[c - nbuf].wait()
            gstart(c).wait()
            out_descs[c] = stage_out(c)
        for c in range(nch - nbuf, nch):
            out_descs[c].wait()

    return gather_k(table, idx_flat)


def _comp_body(t_ref, dt_ref, occ_ref, ox_ref, oy_ref, oz_ref,
               dx_ref, dy_ref, dz_ref, cx_ref, cy_ref, cz_ref):
    oxv = ox_ref[...][0]
    oyv = oy_ref[...][0]
    ozv = oz_ref[...][0]
    dxv = dx_ref[...][0]
    dyv = dy_ref[...][0]
    dzv = dz_ref[...][0]
    norm = jnp.sqrt(dxv * dxv + dyv * dyv + dzv * dzv) + 1e-8
    inv = 1.0 / norm
    ndx, ndy, ndz = dxv * inv, dyv * inv, dzv * inv

    def slab(a, b):
        r = 1.0 / (b * 64.0)
        la = (0.0 - a) * r
        lb = (128.0 - a) * r
        return jnp.minimum(la, lb), jnp.maximum(la, lb)

    lox, hix = slab(oxv * 64.0 + 64.0, ndx)
    loy, hiy = slab(oyv * 64.0 + 64.0, ndy)
    loz, hiz = slab(ozv * 64.0 + 64.0, ndz)
    t_lo = jnp.maximum(jnp.maximum(lox, loy), loz)
    t_hi = jnp.minimum(jnp.minimum(hix, hiy), hiz)

    occ = occ_ref[...][:, 0]
    t3 = t_ref[...].reshape(_NSAMP, 1, 1)
    inside = (t3 >= t_lo) & (t3 <= t_hi)
    sp = jnp.log1p(jnp.exp(occ))
    sigma = jnp.where((occ > _OCC_THRES) & inside, sp, 0.0)
    cum = sigma * dt_ref[...].reshape(_NSAMP, 1, 1)
    k = 1
    while k < _NSAMP:
        z = jnp.zeros((k, 8, 128), jnp.float32)
        cum = cum + jnp.concatenate([z, cum[:-k]], axis=0)
        k *= 2
    e = jnp.exp(-cum)
    e_prev = jnp.concatenate(
        [jnp.ones((1, 8, 128), jnp.float32), e[:-1]], axis=0)
    w = e_prev - e
    wsum = 1.0 - e[_NSAMP - 1]
    tsum = jnp.sum(w * t3, axis=0)

    cx_ref[0] = oxv * wsum + ndx * tsum
    cy_ref[0] = oyv * wsum + ndy * tsum
    cz_ref[0] = ozv * wsum + ndz * tsum


def _composite(t_mid, dt, occ4, ox, oy, oz, dx, dy, dz):
    rv = ox.shape[0]
    tcol_spec = pl.BlockSpec((_NSAMP, 1), lambda b: (0, 0))
    ray_spec = pl.BlockSpec((1, 8, 128), lambda b: (b, 0, 0))
    out_sds = jax.ShapeDtypeStruct((rv, 8, 128), jnp.float32)
    return pl.pallas_call(
        _comp_body,
        grid=(rv,),
        in_specs=[tcol_spec, tcol_spec,
                  pl.BlockSpec((_NSAMP, 1, 8, 128), lambda b: (0, b, 0, 0))]
                 + [ray_spec] * 6,
        out_specs=[ray_spec] * 3,
        out_shape=[out_sds, out_sds, out_sds],
    )(t_mid.reshape(_NSAMP, 1), dt.reshape(_NSAMP, 1),
      occ4, ox, oy, oz, dx, dy, dz)


def kernel(rays_o, rays_d, occ_grid):
    f32 = jnp.float32
    t_edges = jnp.linspace(_NEAR, _FAR, _NSAMP + 1, dtype=f32)
    t_mid = 0.5 * (t_edges[:-1] + t_edges[1:])
    dt = t_edges[1:] - t_edges[:-1]

    ox = rays_o[:, 0].reshape(_RV, 8, 128)
    oy = rays_o[:, 1].reshape(_RV, 8, 128)
    oz = rays_o[:, 2].reshape(_RV, 8, 128)
    dx = rays_d[:, 0].reshape(_RV, 8, 128)
    dy = rays_d[:, 1].reshape(_RV, 8, 128)
    dz = rays_d[:, 2].reshape(_RV, 8, 128)

    h = _RV // 4
    outs = []
    halves = []
    for p in range(4):
        sl = slice(p * h, (p + 1) * h)
        args = (ox[sl], oy[sl], oz[sl], dx[sl], dy[sl], dz[sl])
        idx3 = _compute_idx(t_mid, *args)
        occ = _sc_gather(occ_grid, idx3.reshape(-1))
        halves.append((occ.reshape(_NSAMP, h, 8, 128), args))
    for occ4, args in halves:
        outs.append(_composite(t_mid, dt, occ4, *args))
    cx = jnp.concatenate([o[0] for o in outs]).reshape(-1)
    cy = jnp.concatenate([o[1] for o in outs]).reshape(-1)
    cz = jnp.concatenate([o[2] for o in outs]).reshape(-1)
    return jnp.stack([cx, cy, cz], axis=-1)
